# baseline (speedup 1.0000x reference)
import jax
import jax.numpy as jnp
from jax import lax
from jax.experimental import pallas as pl
from jax.experimental.pallas import tpu as pltpu
from jax.experimental.pallas import tpu_sc as plsc

BATCH = 4096
NDIMS = 4
DIM = 1000
NW = 32
RPW = BATCH // NW
RW = 8
NWAVE = RPW // RW
L = 16
C32 = 32
NF32 = 31
TAIL32 = 968

DMAP = (0, 0, 0, 1, 1, 2)
PICKS = ((0, 1), (1, 2), (3, 2), (2, 3), (4, 3), (5, 3))


def _sc_kernel(xT_hbm, t0_hbm, t1_hbm, t2_hbm, t3_hbm, t4_hbm, t5_hbm,
               bias_hbm, out_hbm, xall_v, b_v, buf_v, svec_v, sem0, sem1):
    wid = lax.axis_index("s") * 2 + lax.axis_index("c")
    base = wid * RPW
    tables = (t0_hbm, t1_hbm, t2_hbm, t3_hbm, t4_hbm, t5_hbm)

    for d in range(NDIMS):
        pltpu.sync_copy(xT_hbm.at[d, pl.ds(base, RPW)],
                        xall_v.at[d, pl.ds(0, RPW)])
    pltpu.sync_copy(bias_hbm, b_v)

    iota = lax.iota(jnp.int32, L)
    tailmask = iota >= 12
    zero16 = jnp.zeros((L,), jnp.float32)

    def issue(w, pb, sem):
        for t in range(6):
            pltpu.async_copy(
                tables[t].at[xall_v.at[DMAP[t], pl.ds(w * RW, RW)]],
                buf_v.at[pb, t], sem)

    def drain(w, pb, sem):
        for t in range(6):
            pltpu.make_async_copy(
                tables[t].at[xall_v.at[DMAP[t], pl.ds(w * RW, RW)]],
                buf_v.at[pb, t], sem).wait()

    def compute(w, pb):
        def unp(v):
            return plsc.unpack(v, format=plsc.PackFormat.INTERLEAVED)

        def chunk(off, accs, masked):
            bb = [unp(b_v[pl.ds(j * DIM + off, C32)]) for j in (1, 2, 3)]
            new = []
            for r in range(RW):
                t6 = [unp(buf_v[pb, t, r, pl.ds(off, C32)]) for t in range(6)]
                acc = list(accs[3 * r:3 * r + 3])
                for h in range(2):
                    e1 = jnp.exp(t6[0][h] + bb[0][h])
                    e2 = jnp.exp(t6[1][h] + t6[3][h] + bb[1][h])
                    e3 = jnp.exp(t6[2][h] + t6[4][h] + t6[5][h] + bb[2][h])
                    if masked:
                        e1 = jnp.where(tailmask, e1, 0.0)
                        e2 = jnp.where(tailmask, e2, 0.0)
                        e3 = jnp.where(tailmask, e3, 0.0)
                    acc = [acc[0] + e1, acc[1] + e2, acc[2] + e3]
                new += acc
            return tuple(new)

        accs = lax.fori_loop(
            0, NF32,
            lambda c, a: chunk(c * C32, a, False),
            tuple([zero16] * (3 * RW)),
        )
        accs = chunk(TAIL32, accs, True)

        for r in range(RW):
            row = w * RW + r
            xsc = [None] * NDIMS
            for j in range(NDIMS):
                xv = xall_v[j, pl.ds(row, L)]
                xsc[j] = xv[0]
            sel = zero16

            def pick(ld, rem32):
                pe, po = plsc.unpack(ld, format=plsc.PackFormat.INTERLEAVED)
                lane = lax.div(rem32, 2)
                vsel = jnp.where(rem32 - 2 * lane == 1, po, pe)
                return jnp.where(iota == lane, vsel, 0.0)

            for t, j in PICKS:
                col = xsc[j]
                rem32 = lax.rem(col, C32)
                sel = sel + pick(buf_v[pb, t, r, pl.ds(col - rem32, C32)],
                                 rem32)
            for j in range(NDIMS):
                off = j * DIM + xsc[j]
                rem32 = lax.rem(off, C32)
                sel = sel + pick(b_v[pl.ds(off - rem32, C32)], rem32)
            selsum = jnp.sum(sel)
            s1s = jnp.sum(accs[3 * r + 0])
            s2s = jnp.sum(accs[3 * r + 1])
            s3s = jnp.sum(accs[3 * r + 2])
            v = jnp.where(iota == 0, selsum,
                          jnp.where(iota == 1, s1s,
                                    jnp.where(iota == 2, s2s,
                                              jnp.where(iota == 3, s3s, 0.0))))
            svec_v[pl.ds(row * L, L)] = v

    issue(0, 0, sem0)

    def body2(i, carry):
        w = 2 * i
        issue(w + 1, 1, sem1)
        drain(w, 0, sem0)
        compute(w, 0)
        issue(w + 2, 0, sem0)
        drain(w + 1, 1, sem1)
        compute(w + 1, 1)
        return carry

    lax.fori_loop(0, NWAVE // 2 - 1, body2, 0)
    issue(NWAVE - 1, 1, sem1)
    drain(NWAVE - 2, 0, sem0)
    compute(NWAVE - 2, 0)
    drain(NWAVE - 1, 1, sem1)
    compute(NWAVE - 1, 1)

    pltpu.sync_copy(svec_v, out_hbm.at[pl.ds(wid * (RPW * L), RPW * L)])


BT = 4096


def _tc_kernel(svecs_ref, b_ref, out_ref):
    sv = svecs_ref[...]
    sel = sv[:, 0:1]
    s1 = sv[:, 1:2]
    s2 = sv[:, 2:3]
    s3 = sv[:, 3:4]
    b0 = b_ref[...][:, 0:DIM]
    s0 = jnp.sum(jnp.exp(b0))
    out_ref[...] = (sel - jnp.log(s0) - jnp.log(s1)
                    - jnp.log(s2) - jnp.log(s3))


@jax.jit
def kernel(x, W, b):
    x32 = x.astype(jnp.int32)
    xT = x32.T
    bf = jnp.bfloat16
    Wb = jax.lax.optimization_barrier(W.astype(bf))
    tabs = [Wb[0:DIM, DIM:2 * DIM], Wb[0:DIM, 2 * DIM:3 * DIM],
            Wb[0:DIM, 3 * DIM:4 * DIM], Wb[DIM:2 * DIM, 2 * DIM:3 * DIM],
            Wb[DIM:2 * DIM, 3 * DIM:4 * DIM],
            Wb[2 * DIM:3 * DIM, 3 * DIM:4 * DIM]]

    mesh = plsc.VectorSubcoreMesh(core_axis_name="c", subcore_axis_name="s")
    svecs = pl.kernel(
        _sc_kernel,
        mesh=mesh,
        out_type=[jax.ShapeDtypeStruct((BATCH * L,), jnp.float32)],
        scratch_types=[
            pltpu.VMEM((NDIMS, RPW + L), jnp.int32),
            pltpu.VMEM((NDIMS * DIM,), jnp.bfloat16),
            pltpu.VMEM((2, 6, RW, DIM), jnp.bfloat16),
            pltpu.VMEM((RPW * L,), jnp.float32),
            pltpu.SemaphoreType.DMA,
            pltpu.SemaphoreType.DMA,
        ],
        compiler_params=pltpu.CompilerParams(use_tc_tiling_on_sc=False,
                                             needs_layout_passes=False),
    )(xT, *tabs, b.astype(bf))[0]

    out = pl.pallas_call(
        _tc_kernel,
        grid=(BATCH // BT,),
        in_specs=[
            pl.BlockSpec((BT, L), lambda i: (i, 0)),
            pl.BlockSpec((1, NDIMS * DIM), lambda i: (0, 0)),
        ],
        out_specs=pl.BlockSpec((BT, 1), lambda i: (i, 0)),
        out_shape=jax.ShapeDtypeStruct((BATCH, 1), jnp.float32),
    )(svecs.reshape(BATCH, L), b.reshape(1, NDIMS * DIM))

    return out.reshape(BATCH)


# speedup vs baseline: 2.1067x; 2.1067x over previous
import jax
import jax.numpy as jnp
from jax import lax
from jax.experimental import pallas as pl
from jax.experimental.pallas import tpu as pltpu
from jax.experimental.pallas import tpu_sc as plsc

BATCH = 4096
NDIMS = 4
DIM = 1000
NW = 32
RPW = BATCH // NW
RW = 8
NWAVE = RPW // RW
L = 16
NFULL = DIM // L
TAIL_OFF = DIM - L

DMAP = (0, 0, 0, 1, 1, 2)
PICKS = ((0, 1), (1, 2), (3, 2), (2, 3), (4, 3), (5, 3))


def _sc_kernel(xT_hbm, t0_hbm, t1_hbm, t2_hbm, t3_hbm, t4_hbm, t5_hbm,
               bias_hbm, out_hbm, xall_v, b_v, buf_v, svec_v, sem0, sem1):
    wid = lax.axis_index("s") * 2 + lax.axis_index("c")
    base = wid * RPW
    tables = (t0_hbm, t1_hbm, t2_hbm, t3_hbm, t4_hbm, t5_hbm)

    for d in range(NDIMS):
        pltpu.sync_copy(xT_hbm.at[d, pl.ds(base, RPW)],
                        xall_v.at[d, pl.ds(0, RPW)])
    pltpu.sync_copy(bias_hbm, b_v)

    iota = lax.iota(jnp.int32, L)
    tailmask = iota >= (L - (DIM - NFULL * L))
    zero16 = jnp.zeros((L,), jnp.float32)

    def issue(w, pb, sem):
        for t in range(6):
            pltpu.async_copy(
                tables[t].at[xall_v.at[DMAP[t], pl.ds(w * RW, RW)]],
                buf_v.at[pb, t], sem)

    def drain(w, pb, sem):
        for t in range(6):
            pltpu.make_async_copy(
                tables[t].at[xall_v.at[DMAP[t], pl.ds(w * RW, RW)]],
                buf_v.at[pb, t], sem).wait()

    def compute(w, pb):
        def chunk(off, accs, masked):
            b1 = b_v[pl.ds(DIM + off, L)]
            b2 = b_v[pl.ds(2 * DIM + off, L)]
            b3 = b_v[pl.ds(3 * DIM + off, L)]
            new = []
            for r in range(RW):
                a1 = buf_v[pb, 0, r, pl.ds(off, L)]
                a2 = buf_v[pb, 1, r, pl.ds(off, L)]
                a3 = buf_v[pb, 2, r, pl.ds(off, L)]
                f2 = buf_v[pb, 3, r, pl.ds(off, L)]
                f3 = buf_v[pb, 4, r, pl.ds(off, L)]
                g3 = buf_v[pb, 5, r, pl.ds(off, L)]
                e1 = jnp.exp(a1 + b1)
                e2 = jnp.exp(a2 + f2 + b2)
                e3 = jnp.exp(a3 + f3 + g3 + b3)
                if masked:
                    e1 = jnp.where(tailmask, e1, 0.0)
                    e2 = jnp.where(tailmask, e2, 0.0)
                    e3 = jnp.where(tailmask, e3, 0.0)
                s1, s2, s3 = accs[3 * r:3 * r + 3]
                new += [s1 + e1, s2 + e2, s3 + e3]
            return tuple(new)

        accs = lax.fori_loop(
            0, NFULL,
            lambda c, a: chunk(c * L, a, False),
            tuple([zero16] * (3 * RW)),
        )
        accs = chunk(TAIL_OFF, accs, True)

        for r in range(RW):
            row = w * RW + r
            xsc = [None] * NDIMS
            for j in range(NDIMS):
                xv = xall_v[j, pl.ds(row, L)]
                xsc[j] = xv[0]
            sel = zero16
            for t, j in PICKS:
                col = xsc[j]
                rem = lax.rem(col, L)
                v = buf_v[pb, t, r, pl.ds(col - rem, L)]
                sel = sel + jnp.where(iota == rem, v, 0.0)
            for j in range(NDIMS):
                off = j * DIM + xsc[j]
                rem = lax.rem(off, L)
                v = b_v[pl.ds(off - rem, L)]
                sel = sel + jnp.where(iota == rem, v, 0.0)
            selsum = jnp.sum(sel)
            s1s = jnp.sum(accs[3 * r + 0])
            s2s = jnp.sum(accs[3 * r + 1])
            s3s = jnp.sum(accs[3 * r + 2])
            v = jnp.where(iota == 0, selsum,
                          jnp.where(iota == 1, s1s,
                                    jnp.where(iota == 2, s2s,
                                              jnp.where(iota == 3, s3s, 0.0))))
            svec_v[pl.ds(row * L, L)] = v

    issue(0, 0, sem0)

    def body2(i, carry):
        w = 2 * i
        issue(w + 1, 1, sem1)
        drain(w, 0, sem0)
        compute(w, 0)
        issue(w + 2, 0, sem0)
        drain(w + 1, 1, sem1)
        compute(w + 1, 1)
        return carry

    lax.fori_loop(0, NWAVE // 2 - 1, body2, 0)
    issue(NWAVE - 1, 1, sem1)
    drain(NWAVE - 2, 0, sem0)
    compute(NWAVE - 2, 0)
    drain(NWAVE - 1, 1, sem1)
    compute(NWAVE - 1, 1)

    pltpu.sync_copy(svec_v, out_hbm.at[pl.ds(wid * (RPW * L), RPW * L)])


BT = 4096


def _tc_kernel(svecs_ref, b_ref, out_ref):
    sv = svecs_ref[...]
    sel = sv[:, 0:1]
    s1 = sv[:, 1:2]
    s2 = sv[:, 2:3]
    s3 = sv[:, 3:4]
    b0 = b_ref[...][:, 0:DIM]
    s0 = jnp.sum(jnp.exp(b0))
    out_ref[...] = (sel - jnp.log(s0) - jnp.log(s1)
                    - jnp.log(s2) - jnp.log(s3))


@jax.jit
def kernel(x, W, b):
    x32 = x.astype(jnp.int32)
    xT = x32.T
    tabs = [W[0:DIM, DIM:2 * DIM], W[0:DIM, 2 * DIM:3 * DIM],
            W[0:DIM, 3 * DIM:4 * DIM], W[DIM:2 * DIM, 2 * DIM:3 * DIM],
            W[DIM:2 * DIM, 3 * DIM:4 * DIM],
            W[2 * DIM:3 * DIM, 3 * DIM:4 * DIM]]

    mesh = plsc.VectorSubcoreMesh(core_axis_name="c", subcore_axis_name="s")
    svecs = pl.kernel(
        _sc_kernel,
        mesh=mesh,
        out_type=[jax.ShapeDtypeStruct((BATCH * L,), jnp.float32)],
        scratch_types=[
            pltpu.VMEM((NDIMS, RPW + L), jnp.int32),
            pltpu.VMEM((NDIMS * DIM,), jnp.float32),
            pltpu.VMEM((2, 6, RW, DIM), jnp.float32),
            pltpu.VMEM((RPW * L,), jnp.float32),
            pltpu.SemaphoreType.DMA,
            pltpu.SemaphoreType.DMA,
        ],
        compiler_params=pltpu.CompilerParams(use_tc_tiling_on_sc=False,
                                             needs_layout_passes=False),
    )(xT, *tabs, b)[0]

    out = pl.pallas_call(
        _tc_kernel,
        grid=(BATCH // BT,),
        in_specs=[
            pl.BlockSpec((BT, L), lambda i: (i, 0)),
            pl.BlockSpec((1, NDIMS * DIM), lambda i: (0, 0)),
        ],
        out_specs=pl.BlockSpec((BT, 1), lambda i: (i, 0)),
        out_shape=jax.ShapeDtypeStruct((BATCH, 1), jnp.float32),
    )(svecs.reshape(BATCH, L), b.reshape(1, NDIMS * DIM))

    return out.reshape(BATCH)
